# baseline (speedup 1.0000x reference)
import functools

import jax
import jax.numpy as jnp
from jax.experimental import pallas as pl
from jax.experimental.pallas import tpu as pltpu

_EPS = 1e-5
_ROWS = 200


def _bn_relu(y, g, b):
    return jnp.maximum(g * (y * (1.0 / jnp.sqrt(1.0 + _EPS))) + b, 0.0)


def _fused_body(nblk, adja_ref, adjb_ref, x_ref, w1_ref, b1_ref, g_ref,
                be_ref, w2_ref, b2_ref, m1w_ref, m1b_ref, m1g_ref, m1be_ref,
                m2w_ref, m2b_ref, m2g_ref, m2be_ref, m3w_ref, m3b_ref,
                out_ref, s2_ref):
    i = pl.program_id(0)
    even = (i % 2) == 0

    def pass1(aref):
        def f():
            s1 = jnp.dot(x_ref[...], w1_ref[...],
                         preferred_element_type=jnp.float32)
            y = jnp.dot(aref[...], s1, preferred_element_type=jnp.float32)
            x1 = _bn_relu(y + b1_ref[...], g_ref[...], be_ref[...])
            s2_ref[pl.ds(i * _ROWS, _ROWS), :] = jnp.dot(
                x1, w2_ref[...], preferred_element_type=jnp.float32)
        return f

    def pass2(aref):
        def f():
            y = jnp.dot(aref[...], s2_ref[...],
                        preferred_element_type=jnp.float32)
            x2 = _bn_relu(y + b2_ref[...], g_ref[...], be_ref[...])
            h = _bn_relu(jnp.dot(x2, m1w_ref[...],
                                 preferred_element_type=jnp.float32)
                         + m1b_ref[...], m1g_ref[...], m1be_ref[...])
            h = _bn_relu(jnp.dot(h, m2w_ref[...],
                                 preferred_element_type=jnp.float32)
                         + m2b_ref[...], m2g_ref[...], m2be_ref[...])
            o = jnp.dot(h, m3w_ref[...],
                        preferred_element_type=jnp.float32) + m3b_ref[...]
            m = jnp.max(o, axis=1, keepdims=True)
            lse = jnp.log(jnp.sum(jnp.exp(o - m), axis=1, keepdims=True)) + m
            out_ref[...] = o - lse
        return f

    pl.when((i < nblk) & even)(pass1(adja_ref))
    pl.when((i < nblk) & jnp.logical_not(even))(pass1(adjb_ref))
    pl.when((i >= nblk) & even)(pass2(adja_ref))
    pl.when((i >= nblk) & jnp.logical_not(even))(pass2(adjb_ref))


def _const_spec(shape):
    return pl.BlockSpec(shape, lambda i: (0,) * len(shape))


def kernel(x, adj, W1, b1, W2, b2, bn1_g, bn1_b, m1_W, m1_b, m1_g, m1_be,
           m2_W, m2_b, m2_g, m2_be, m3_W, m3_b):
    n, nfeat = x.shape
    nhid = W1.shape[1]
    nmid = m1_W.shape[1]
    nclass = m3_W.shape[1]
    f32 = jnp.float32
    nblk = n // _ROWS

    def row(v):
        return v.reshape(1, -1)

    def tile(s):
        return jnp.where(s < nblk, s, s - nblk)

    def adja_map(i):
        u = jnp.minimum(((i + 1) // 2) * 2, 2 * nblk - 2)
        return (tile(u), 0)

    def adjb_map(i):
        v = jnp.minimum((i // 2) * 2 + 1, 2 * nblk - 1)
        return (tile(v), 0)

    def out_map(i):
        return (jnp.maximum(i - nblk, 0), 0)

    body = functools.partial(_fused_body, nblk)

    out = pl.pallas_call(
        body,
        grid=(2 * nblk,),
        in_specs=[pl.BlockSpec((_ROWS, n), adja_map),
                  pl.BlockSpec((_ROWS, n), adjb_map),
                  _const_spec((n, nfeat)),
                  _const_spec((nfeat, nhid)), _const_spec((1, nhid)),
                  _const_spec((1, nhid)), _const_spec((1, nhid)),
                  _const_spec((nhid, nhid)), _const_spec((1, nhid)),
                  _const_spec((nhid, nmid)), _const_spec((1, nmid)),
                  _const_spec((1, nmid)), _const_spec((1, nmid)),
                  _const_spec((nmid, nhid)), _const_spec((1, nhid)),
                  _const_spec((1, nhid)), _const_spec((1, nhid)),
                  _const_spec((nhid, nclass)), _const_spec((1, nclass))],
        out_specs=pl.BlockSpec((_ROWS, nclass), out_map),
        out_shape=jax.ShapeDtypeStruct((n, nclass), f32),
        scratch_shapes=[pltpu.VMEM((n, nhid), f32)],
        compiler_params=pltpu.CompilerParams(
            dimension_semantics=("arbitrary",)),
    )(adj, adj, x, W1, row(b1), row(bn1_g), row(bn1_b), W2, row(b2),
      m1_W, row(m1_b), row(m1_g), row(m1_be),
      m2_W, row(m2_b), row(m2_g), row(m2_be),
      m3_W, row(m3_b))
    return out


# speedup vs baseline: 1.1669x; 1.1669x over previous
import functools

import jax
import jax.numpy as jnp
from jax.experimental import pallas as pl
from jax.experimental.pallas import tpu as pltpu

_EPS = 1e-5
_ROWS = 200


def _bn_relu(y, g, b):
    return jnp.maximum(g * (y * (1.0 / jnp.sqrt(1.0 + _EPS))) + b, 0.0)


def _outer_body(nblk, n, adj_ref, x_ref, w1_ref, b1_ref, g_ref, be_ref,
                w2_ref, b2_ref, m1w_ref, m1b_ref, m1g_ref, m1be_ref,
                m2w_ref, m2b_ref, m2g_ref, m2be_ref, m3w_ref, m3b_ref,
                out_ref, s2_ref):
    s1 = jnp.dot(x_ref[...], w1_ref[...], preferred_element_type=jnp.float32)

    def step(idx, adj_blk_ref):
        j = idx[0]

        @pl.when(j < nblk)
        def _pass1():
            y = jnp.dot(adj_blk_ref[...], s1,
                        preferred_element_type=jnp.float32)
            x1 = _bn_relu(y + b1_ref[...], g_ref[...], be_ref[...])
            s2_ref[pl.ds(j * _ROWS, _ROWS), :] = jnp.dot(
                x1, w2_ref[...], preferred_element_type=jnp.float32)

        @pl.when(j >= nblk)
        def _pass2():
            y = jnp.dot(adj_blk_ref[...], s2_ref[...],
                        preferred_element_type=jnp.float32)
            x2 = _bn_relu(y + b2_ref[...], g_ref[...], be_ref[...])
            h = _bn_relu(jnp.dot(x2, m1w_ref[...],
                                 preferred_element_type=jnp.float32)
                         + m1b_ref[...], m1g_ref[...], m1be_ref[...])
            h = _bn_relu(jnp.dot(h, m2w_ref[...],
                                 preferred_element_type=jnp.float32)
                         + m2b_ref[...], m2g_ref[...], m2be_ref[...])
            o = jnp.dot(h, m3w_ref[...],
                        preferred_element_type=jnp.float32) + m3b_ref[...]
            m = jnp.max(o, axis=1, keepdims=True)
            lse = jnp.log(jnp.sum(jnp.exp(o - m), axis=1, keepdims=True)) + m
            out_ref[pl.ds((j - nblk) * _ROWS, _ROWS), :] = o - lse

    pltpu.emit_pipeline(
        step,
        grid=(2 * nblk,),
        in_specs=[pl.BlockSpec(
            (_ROWS, n),
            lambda j: (jax.lax.select(j < nblk, j, j - nblk), 0),
            pipeline_mode=pl.Buffered(buffer_count=5, use_lookahead=True))],
        _explicit_indices=True,
    )(adj_ref)


def _vmem_spec(shape):
    return pl.BlockSpec(shape, lambda: (0,) * len(shape),
                        memory_space=pltpu.MemorySpace.VMEM)


def kernel(x, adj, W1, b1, W2, b2, bn1_g, bn1_b, m1_W, m1_b, m1_g, m1_be,
           m2_W, m2_b, m2_g, m2_be, m3_W, m3_b):
    n, nfeat = x.shape
    nhid = W1.shape[1]
    nmid = m1_W.shape[1]
    nclass = m3_W.shape[1]
    f32 = jnp.float32
    nblk = n // _ROWS

    def row(v):
        return v.reshape(1, -1)

    body = functools.partial(_outer_body, nblk, n)

    out = pl.pallas_call(
        body,
        in_specs=[pl.BlockSpec(memory_space=pltpu.MemorySpace.HBM),
                  _vmem_spec((n, nfeat)),
                  _vmem_spec((nfeat, nhid)), _vmem_spec((1, nhid)),
                  _vmem_spec((1, nhid)), _vmem_spec((1, nhid)),
                  _vmem_spec((nhid, nhid)), _vmem_spec((1, nhid)),
                  _vmem_spec((nhid, nmid)), _vmem_spec((1, nmid)),
                  _vmem_spec((1, nmid)), _vmem_spec((1, nmid)),
                  _vmem_spec((nmid, nhid)), _vmem_spec((1, nhid)),
                  _vmem_spec((1, nhid)), _vmem_spec((1, nhid)),
                  _vmem_spec((nhid, nclass)), _vmem_spec((1, nclass))],
        out_specs=_vmem_spec((n, nclass)),
        out_shape=jax.ShapeDtypeStruct((n, nclass), f32),
        scratch_shapes=[pltpu.VMEM((n, nhid), f32)],
    )(adj, x, W1, row(b1), row(bn1_g), row(bn1_b), W2, row(b2),
      m1_W, row(m1_b), row(m1_g), row(m1_be),
      m2_W, row(m2_b), row(m2_g), row(m2_be),
      m3_W, row(m3_b))
    return out
